# Initial kernel scaffold; baseline (speedup 1.0000x reference)
import jax
import jax.numpy as jnp
from jax.experimental import pallas as pl


def kernel(state, U):
    raise NotImplementedError("write your pallas kernel here")


# speedup vs baseline: 295.2605x; 295.2605x over previous
import jax
import jax.numpy as jnp
from jax.experimental import pallas as pl
from jax.experimental.pallas import tpu as pltpu

NQ = 23
DIM = 1 << NQ
B = 1 << 17
G = DIM // B
H = G // 2
Q = G // 4


def _body(u_ref, a_ref, b_ref, o_ref):
    i = pl.program_id(0)
    k = jnp.maximum(i - H, 0)
    which = k % 2
    blend = i >= H
    c0 = jnp.where(blend, jnp.where(which == 0, u_ref[0, 0], u_ref[1, 0]), 1.0)
    c1 = jnp.where(blend, jnp.where(which == 0, u_ref[0, 1], u_ref[1, 1]), 0.0)
    o_ref[...] = c0 * a_ref[...] + c1 * b_ref[...]


def _a_map(i):
    k = jnp.maximum(i - H, 0)
    return (jnp.where(i < H, i, H + k // 2),)


def _b_map(i):
    k = jnp.maximum(i - H, 0)
    return (jnp.where(i < H, 0, H + Q + k // 2),)


def _o_map(i):
    k = jnp.maximum(i - H, 0)
    return (jnp.where(i < H, i, H + (k % 2) * Q + k // 2),)


def kernel(state, U):
    return pl.pallas_call(
        _body,
        grid=(G,),
        in_specs=[
            pl.BlockSpec(memory_space=pltpu.SMEM),
            pl.BlockSpec((B,), _a_map),
            pl.BlockSpec((B,), _b_map),
        ],
        out_specs=pl.BlockSpec((B,), _o_map),
        out_shape=jax.ShapeDtypeStruct((DIM,), jnp.float32),
    )(U, state, state)


# speedup vs baseline: 411.6332x; 1.3941x over previous
import jax
import jax.numpy as jnp
from jax.experimental import pallas as pl
from jax.experimental.pallas import tpu as pltpu

NQ = 23
DIM = 1 << NQ
B = 1 << 18
G = DIM // B
H = G // 2
Q = G // 4


def _body(u_ref, a_ref, b_ref, o_ref):
    i = pl.program_id(0)
    k = jnp.maximum(i - H, 0)
    which = k % 2
    blend = i >= H
    c0 = jnp.where(blend, jnp.where(which == 0, u_ref[0, 0], u_ref[1, 0]), 1.0)
    c1 = jnp.where(blend, jnp.where(which == 0, u_ref[0, 1], u_ref[1, 1]), 0.0)
    o_ref[...] = c0 * a_ref[...] + c1 * b_ref[...]


def _a_map(i):
    k = jnp.maximum(i - H, 0)
    return (jnp.where(i < H, i, H + k // 2),)


def _b_map(i):
    k = jnp.maximum(i - H, 0)
    return (jnp.where(i < H, 0, H + Q + k // 2),)


def _o_map(i):
    k = jnp.maximum(i - H, 0)
    return (jnp.where(i < H, i, H + (k % 2) * Q + k // 2),)


def kernel(state, U):
    return pl.pallas_call(
        _body,
        grid=(G,),
        in_specs=[
            pl.BlockSpec(memory_space=pltpu.SMEM),
            pl.BlockSpec((B,), _a_map),
            pl.BlockSpec((B,), _b_map),
        ],
        out_specs=pl.BlockSpec((B,), _o_map),
        out_shape=jax.ShapeDtypeStruct((DIM,), jnp.float32),
    )(U, state, state)


# speedup vs baseline: 531.5038x; 1.2912x over previous
import jax
import jax.numpy as jnp
from jax.experimental import pallas as pl
from jax.experimental.pallas import tpu as pltpu

NQ = 23
DIM = 1 << NQ
B = 1 << 19
G = DIM // B
H = G // 2
Q = G // 4


def _body(u_ref, a_ref, b_ref, o_ref):
    i = pl.program_id(0)
    k = jnp.maximum(i - H, 0)
    which = k % 2
    blend = i >= H
    c0 = jnp.where(blend, jnp.where(which == 0, u_ref[0, 0], u_ref[1, 0]), 1.0)
    c1 = jnp.where(blend, jnp.where(which == 0, u_ref[0, 1], u_ref[1, 1]), 0.0)
    o_ref[...] = c0 * a_ref[...] + c1 * b_ref[...]


def _a_map(i):
    k = jnp.maximum(i - H, 0)
    return (jnp.where(i < H, i, H + k // 2),)


def _b_map(i):
    k = jnp.maximum(i - H, 0)
    return (jnp.where(i < H, 0, H + Q + k // 2),)


def _o_map(i):
    k = jnp.maximum(i - H, 0)
    return (jnp.where(i < H, i, H + (k % 2) * Q + k // 2),)


def kernel(state, U):
    return pl.pallas_call(
        _body,
        grid=(G,),
        in_specs=[
            pl.BlockSpec(memory_space=pltpu.SMEM),
            pl.BlockSpec((B,), _a_map),
            pl.BlockSpec((B,), _b_map),
        ],
        out_specs=pl.BlockSpec((B,), _o_map),
        out_shape=jax.ShapeDtypeStruct((DIM,), jnp.float32),
    )(U, state, state)


# speedup vs baseline: 597.0210x; 1.1233x over previous
import jax
import jax.numpy as jnp
from jax.experimental import pallas as pl
from jax.experimental.pallas import tpu as pltpu

NQ = 23
DIM = 1 << NQ
B = 1 << 20
G = DIM // B
H = G // 2
Q = G // 4


def _body(u_ref, a_ref, b_ref, o_ref):
    i = pl.program_id(0)
    k = jnp.maximum(i - H, 0)
    which = k % 2
    blend = i >= H
    c0 = jnp.where(blend, jnp.where(which == 0, u_ref[0, 0], u_ref[1, 0]), 1.0)
    c1 = jnp.where(blend, jnp.where(which == 0, u_ref[0, 1], u_ref[1, 1]), 0.0)
    o_ref[...] = c0 * a_ref[...] + c1 * b_ref[...]


def _a_map(i):
    k = jnp.maximum(i - H, 0)
    return (jnp.where(i < H, i, H + k // 2),)


def _b_map(i):
    k = jnp.maximum(i - H, 0)
    return (jnp.where(i < H, 0, H + Q + k // 2),)


def _o_map(i):
    k = jnp.maximum(i - H, 0)
    return (jnp.where(i < H, i, H + (k % 2) * Q + k // 2),)


def kernel(state, U):
    return pl.pallas_call(
        _body,
        grid=(G,),
        in_specs=[
            pl.BlockSpec(memory_space=pltpu.SMEM),
            pl.BlockSpec((B,), _a_map),
            pl.BlockSpec((B,), _b_map),
        ],
        out_specs=pl.BlockSpec((B,), _o_map),
        out_shape=jax.ShapeDtypeStruct((DIM,), jnp.float32),
    )(U, state, state)


# speedup vs baseline: 613.2994x; 1.0273x over previous
import jax
import jax.numpy as jnp
from jax.experimental import pallas as pl
from jax.experimental.pallas import tpu as pltpu

NQ = 23
DIM = 1 << NQ
B = 1 << 21
G = DIM // B
H = G // 2
Q = G // 4


def _body(u_ref, a_ref, b_ref, o_ref):
    i = pl.program_id(0)
    k = jnp.maximum(i - H, 0)
    which = k % 2
    blend = i >= H
    c0 = jnp.where(blend, jnp.where(which == 0, u_ref[0, 0], u_ref[1, 0]), 1.0)
    c1 = jnp.where(blend, jnp.where(which == 0, u_ref[0, 1], u_ref[1, 1]), 0.0)
    o_ref[...] = c0 * a_ref[...] + c1 * b_ref[...]


def _a_map(i):
    k = jnp.maximum(i - H, 0)
    return (jnp.where(i < H, i, H + k // 2),)


def _b_map(i):
    k = jnp.maximum(i - H, 0)
    return (jnp.where(i < H, 0, H + Q + k // 2),)


def _o_map(i):
    k = jnp.maximum(i - H, 0)
    return (jnp.where(i < H, i, H + (k % 2) * Q + k // 2),)


def kernel(state, U):
    return pl.pallas_call(
        _body,
        grid=(G,),
        in_specs=[
            pl.BlockSpec(memory_space=pltpu.SMEM),
            pl.BlockSpec((B,), _a_map),
            pl.BlockSpec((B,), _b_map),
        ],
        out_specs=pl.BlockSpec((B,), _o_map),
        out_shape=jax.ShapeDtypeStruct((DIM,), jnp.float32),
    )(U, state, state)
